# baseline (speedup 1.0000x reference)
import jax
import jax.numpy as jnp
from jax.experimental import pallas as pl
from jax.experimental.pallas import tpu as pltpu

_BM = 400


def _gcn_body(x_ref, w_ref, b_ref, adj_ref, out_ref, hidden_ref):
    i = pl.program_id(0)

    @pl.when(i == 0)
    def _compute_hidden():
        hidden_ref[...] = (
            jax.lax.dot_general(
                x_ref[...], w_ref[...],
                dimension_numbers=(((1,), (1,)), ((), ())),
                preferred_element_type=jnp.float32,
            )
            + b_ref[...]
        ).astype(jnp.bfloat16)

    out_ref[...] = jnp.maximum(
        jnp.dot(adj_ref[...].astype(jnp.bfloat16), hidden_ref[...],
                preferred_element_type=jnp.float32),
        0.0,
    )


def kernel(x, adj, W, b):
    n, d_in = x.shape
    d_out = W.shape[0]
    out = pl.pallas_call(
        _gcn_body,
        grid=(n // _BM,),
        in_specs=[
            pl.BlockSpec((n, d_in), lambda i: (0, 0)),
            pl.BlockSpec((d_out, d_in), lambda i: (0, 0)),
            pl.BlockSpec((1, d_out), lambda i: (0, 0)),
            pl.BlockSpec((_BM, n), lambda i: (i, 0)),
        ],
        out_specs=pl.BlockSpec((_BM, d_out), lambda i: (i, 0)),
        out_shape=jax.ShapeDtypeStruct((n, d_out), jnp.float32),
        scratch_shapes=[pltpu.VMEM((n, d_out), jnp.bfloat16)],
    )(x, W, b.reshape(1, d_out), adj)
    return out, adj


# speedup vs baseline: 1.4712x; 1.4712x over previous
import jax
import jax.numpy as jnp
from jax.experimental import pallas as pl
from jax.experimental.pallas import tpu as pltpu

_BM = 200


def _gcn_body(x_ref, w_ref, b_ref, adj_ref, out_ref, adj_out_ref, hidden_ref):
    i = pl.program_id(0)

    @pl.when(i == 0)
    def _compute_hidden():
        hidden_ref[...] = (
            jax.lax.dot_general(
                x_ref[...], w_ref[...],
                dimension_numbers=(((1,), (1,)), ((), ())),
                preferred_element_type=jnp.float32,
            )
            + b_ref[...]
        ).astype(jnp.bfloat16)

    out_ref[...] = jnp.maximum(
        jnp.dot(adj_ref[...].astype(jnp.bfloat16), hidden_ref[...],
                preferred_element_type=jnp.float32),
        0.0,
    )
    adj_out_ref[...] = adj_ref[...]


def kernel(x, adj, W, b):
    n, d_in = x.shape
    d_out = W.shape[0]
    out, adj_out = pl.pallas_call(
        _gcn_body,
        grid=(n // _BM,),
        in_specs=[
            pl.BlockSpec((n, d_in), lambda i: (0, 0)),
            pl.BlockSpec((d_out, d_in), lambda i: (0, 0)),
            pl.BlockSpec((1, d_out), lambda i: (0, 0)),
            pl.BlockSpec((_BM, n), lambda i: (i, 0)),
        ],
        out_specs=[
            pl.BlockSpec((_BM, d_out), lambda i: (i, 0)),
            pl.BlockSpec((_BM, n), lambda i: (i, 0)),
        ],
        out_shape=[
            jax.ShapeDtypeStruct((n, d_out), jnp.float32),
            jax.ShapeDtypeStruct((n, n), jnp.float32),
        ],
        scratch_shapes=[pltpu.VMEM((n, d_out), jnp.bfloat16)],
    )(x, W, b.reshape(1, d_out), adj)
    return out, adj_out
